# baseline (speedup 1.0000x reference)
import functools

import jax
import jax.numpy as jnp
from jax import lax
from jax.experimental import pallas as pl
from jax.experimental.pallas import tpu as pltpu
from jax.experimental.pallas import tpu_sc as plsc


def _make_sc_partials(B, CTX, NEG, D, NW):
    assert D == 64
    BW = B // NW
    CB = 4
    NCH = BW // CB
    UR = CB * CTX
    NR = CB * NEG

    mesh = plsc.VectorSubcoreMesh(core_axis_name="c", subcore_axis_name="s")
    nw = mesh.num_cores * mesh.num_subcores
    assert nw == NW

    @functools.partial(
        pl.kernel,
        mesh=mesh,
        out_type=[
            jax.ShapeDtypeStruct((B * 16,), jnp.float32),
            jax.ShapeDtypeStruct((B * NEG * 16,), jnp.float32),
        ],
        scratch_types=[
            pltpu.VMEM((BW * CTX + UR,), jnp.int32),
            pltpu.VMEM((BW * NEG + NR,), jnp.int32),
            pltpu.VMEM((BW + 32,), jnp.int32),
            pltpu.VMEM((2, UR, 2 * D), jnp.float32),
            pltpu.VMEM((2, NR, 2 * D), jnp.float32),
            pltpu.VMEM((2 * CB, 2 * D), jnp.float32),
            pltpu.VMEM((BW * 16,), jnp.float32),
            pltpu.VMEM((NR * 16,), jnp.float32),
            pltpu.SemaphoreType.DMA,
            pltpu.SemaphoreType.DMA,
            pltpu.SemaphoreType.DMA,
        ],
    )
    def sc_partials(u_hbm, v_hbm, posu_hbm, posv_hbm, negv_hbm,
                    pos_out, neg_out,
                    posu_idx, negv_idx, posv_idx,
                    u_rows, n_rows, pv_rows, pos_part, neg_part,
                    sem0, sem1, semp):
        wid = lax.axis_index("s") * mesh.num_cores + lax.axis_index("c")
        base = wid * BW
        pltpu.sync_copy(posu_hbm.at[pl.ds(base * CTX, BW * CTX)],
                        posu_idx.at[pl.ds(0, BW * CTX)])
        pltpu.sync_copy(negv_hbm.at[pl.ds(base * NEG, BW * NEG)],
                        negv_idx.at[pl.ds(0, BW * NEG)])
        pltpu.sync_copy(posv_hbm.at[pl.ds(base, BW)],
                        posv_idx.at[pl.ds(0, BW)])
        zeros16 = jnp.zeros((16,), jnp.int32)
        for k in range(UR // 16):
            posu_idx[BW * CTX + k * 16:BW * CTX + (k + 1) * 16] = zeros16
        for k in range(NR // 16):
            negv_idx[BW * NEG + k * 16:BW * NEG + (k + 1) * 16] = zeros16
        for k in range(2):
            posv_idx[BW + k * 16:BW + (k + 1) * 16] = zeros16

        sems = [sem0, sem1]

        def fire(c, d):
            pltpu.async_copy(
                u_hbm.at[posu_idx.at[pl.ds(c * UR, UR)]], u_rows.at[d], sems[d])
            pltpu.async_copy(
                v_hbm.at[negv_idx.at[pl.ds(c * NR, NR)]], n_rows.at[d], sems[d])

        def fire_pv(i2):
            pltpu.async_copy(
                v_hbm.at[posv_idx.at[pl.ds(i2 * 2 * CB, 2 * CB)]],
                pv_rows, semp)

        def drain(d):
            pltpu.make_async_copy(
                u_hbm.at[pl.ds(0, UR), :], u_rows.at[d], sems[d]).wait()
            pltpu.make_async_copy(
                v_hbm.at[pl.ds(0, NR), :], n_rows.at[d], sems[d]).wait()

        def drain_pv():
            pltpu.make_async_copy(
                v_hbm.at[pl.ds(0, 2 * CB), :], pv_rows, semp).wait()

        def compute(c, d):
            for b in range(CB):
                acc = [u_rows[d, b * CTX, j * 16:(j + 1) * 16]
                       for j in range(4)]
                for r in range(1, CTX):
                    for j in range(4):
                        acc[j] = acc[j] + u_rows[d, b * CTX + r,
                                                 j * 16:(j + 1) * 16]
                t = acc[0] * pv_rows[d * CB + b, 0:16]
                for j in range(1, 4):
                    t = t + acc[j] * pv_rows[d * CB + b, j * 16:(j + 1) * 16]
                pos_part[pl.ds((c * CB + b) * 16, 16)] = t
                for n in range(NEG):
                    row = b * NEG + n
                    t2 = acc[0] * n_rows[d, row, 0:16]
                    for j in range(1, 4):
                        t2 = t2 + acc[j] * n_rows[d, row, j * 16:(j + 1) * 16]
                    neg_part[row * 16:(row + 1) * 16] = t2
            pltpu.sync_copy(
                neg_part, neg_out.at[pl.ds((base * NEG + c * NR) * 16, NR * 16)])

        fire(0, 0)
        fire_pv(0)

        def body(i2, carry):
            c0 = i2 * 2
            drain(0)
            drain_pv()
            fire(c0 + 1, 1)
            compute(c0, 0)
            drain(1)
            fire(c0 + 2, 0)
            compute(c0 + 1, 1)
            return carry

        def outer(i2, carry):
            carry = body(i2, carry)
            fire_pv(i2 + 1)
            return carry

        lax.fori_loop(0, NCH // 2 - 1, outer, 0)
        body(NCH // 2 - 1, 0)
        pltpu.make_async_copy(
            u_hbm.at[pl.ds(0, UR), :], u_rows.at[0], sems[0]).wait()
        pltpu.make_async_copy(
            v_hbm.at[pl.ds(0, NR), :], n_rows.at[0], sems[0]).wait()
        pltpu.sync_copy(pos_part, pos_out.at[pl.ds(base * 16, BW * 16)])

    return sc_partials


def _make_loss_kernel(n_blocks):
    def loss_kernel(scale_ref, pos_ref, neg_ref, out_ref):
        i = pl.program_id(0)
        inv_len = scale_ref[0]
        rows = lax.broadcasted_iota(jnp.int32, (128, 8), 0)
        cols = lax.broadcasted_iota(jnp.int32, (128, 8), 1)
        fold = jnp.where(rows // 16 == cols, 1.0, 0.0).astype(jnp.float32)

        def logsig(x):
            return jnp.minimum(x, 0.0) - jnp.log1p(jnp.exp(-jnp.abs(x)))

        p = jax.lax.dot(pos_ref[...], fold) * inv_len
        n = jax.lax.dot(neg_ref[...], fold) * inv_len
        part = jnp.sum(logsig(p)) + jnp.sum(logsig(-n))

        @pl.when(i == 0)
        def _():
            out_ref[...] = jnp.zeros((1, 1), jnp.float32)
        out_ref[...] += part[None, None]

    return loss_kernel


def kernel(u_table, v_table, pos_u, pos_v, neg_v, length, embedding_dim):
    B, CTX = pos_u.shape
    NEG = neg_v.shape[1]
    D = u_table.shape[1]
    NW = 32

    posu_flat = jnp.maximum(pos_u.astype(jnp.int32), 0).reshape(-1)
    negv_flat = jnp.maximum(neg_v.astype(jnp.int32), 0).reshape(-1)
    posv = jnp.maximum(pos_v.astype(jnp.int32), 0)

    u_pad = jnp.concatenate([u_table, u_table], axis=1)
    v_pad = jnp.concatenate([v_table, v_table], axis=1)

    sc_partials = _make_sc_partials(B, CTX, NEG, D, NW)
    pos_part, neg_part = sc_partials(u_pad, v_pad, posu_flat, posv, negv_flat)

    pos2d = pos_part.reshape(B * 16 // 128, 128)
    neg2d = neg_part.reshape(B * NEG * 16 // 128, 128)
    GRID = 8
    rp = pos2d.shape[0] // GRID
    rn = neg2d.shape[0] // GRID

    inv_len = (1.0 / jnp.asarray(length, jnp.float32)).reshape(1)

    total = pl.pallas_call(
        _make_loss_kernel(GRID),
        grid=(GRID,),
        in_specs=[
            pl.BlockSpec(memory_space=pltpu.SMEM),
            pl.BlockSpec((rp, 128), lambda i: (i, 0)),
            pl.BlockSpec((rn, 128), lambda i: (i, 0)),
        ],
        out_specs=pl.BlockSpec((1, 1), lambda i: (0, 0)),
        out_shape=jax.ShapeDtypeStruct((1, 1), jnp.float32),
    )(inv_len, pos2d, neg2d)

    return (-total[0, 0]) / jnp.asarray(embedding_dim, jnp.float32)


# speedup vs baseline: 1.5240x; 1.5240x over previous
import functools

import jax
import jax.numpy as jnp
from jax import lax
from jax.experimental import pallas as pl
from jax.experimental.pallas import tpu as pltpu
from jax.experimental.pallas import tpu_sc as plsc


def _make_sc_partials(B, CTX, NEG, D, NW):
    assert D == 64
    BW = B // NW
    CB = 4
    NCH = BW // CB
    UR = CB * CTX
    NR = CB * NEG

    mesh = plsc.VectorSubcoreMesh(core_axis_name="c", subcore_axis_name="s")
    nw = mesh.num_cores * mesh.num_subcores
    assert nw == NW

    @functools.partial(
        pl.kernel,
        mesh=mesh,
        out_type=[
            jax.ShapeDtypeStruct((B * 16,), jnp.float32),
            jax.ShapeDtypeStruct((B * NEG * 16,), jnp.float32),
        ],
        scratch_types=[
            pltpu.VMEM((BW * CTX + UR,), jnp.int32),
            pltpu.VMEM((BW * NEG + NR,), jnp.int32),
            pltpu.VMEM((BW + 32,), jnp.int32),
            pltpu.VMEM((2, UR, D), jnp.float32),
            pltpu.VMEM((2, NR, D), jnp.float32),
            pltpu.VMEM((2 * CB, D), jnp.float32),
            pltpu.VMEM((BW * 16,), jnp.float32),
            pltpu.VMEM((NR * 16,), jnp.float32),
            pltpu.SemaphoreType.DMA,
            pltpu.SemaphoreType.DMA,
            pltpu.SemaphoreType.DMA,
        ],
    )
    def sc_partials(u_hbm, v_hbm, posu_hbm, posv_hbm, negv_hbm,
                    pos_out, neg_out,
                    posu_idx, negv_idx, posv_idx,
                    u_rows, n_rows, pv_rows, pos_part, neg_part,
                    sem0, sem1, semp):
        wid = lax.axis_index("s") * mesh.num_cores + lax.axis_index("c")
        base = wid * BW
        pltpu.sync_copy(posu_hbm.at[pl.ds(base * CTX, BW * CTX)],
                        posu_idx.at[pl.ds(0, BW * CTX)])
        pltpu.sync_copy(negv_hbm.at[pl.ds(base * NEG, BW * NEG)],
                        negv_idx.at[pl.ds(0, BW * NEG)])
        pltpu.sync_copy(posv_hbm.at[pl.ds(base, BW)],
                        posv_idx.at[pl.ds(0, BW)])
        zeros16 = jnp.zeros((16,), jnp.int32)
        for k in range(UR // 16):
            posu_idx[BW * CTX + k * 16:BW * CTX + (k + 1) * 16] = zeros16
        for k in range(NR // 16):
            negv_idx[BW * NEG + k * 16:BW * NEG + (k + 1) * 16] = zeros16
        for k in range(2):
            posv_idx[BW + k * 16:BW + (k + 1) * 16] = zeros16

        sems = [sem0, sem1]

        def fire(c, d):
            uidxv = [posu_idx[pl.ds(c * UR + k * 16, 16)]
                     for k in range(UR // 16)]
            nidxv = [negv_idx[pl.ds(c * NR + k * 16, 16)]
                     for k in range(NR // 16)]
            for k in range(UR // 16):
                for l in range(16):
                    fp = k * 16 + l
                    pltpu.async_copy(
                        u_hbm.at[pl.ds(uidxv[k][l], 1), :],
                        u_rows.at[d, pl.ds(fp, 1), :], sems[d])
                    pltpu.async_copy(
                        v_hbm.at[pl.ds(nidxv[k][l], 1), :],
                        n_rows.at[d, pl.ds(fp, 1), :], sems[d])

        def fire_pv(i2):
            pvidxv = posv_idx[pl.ds(i2 * 2 * CB, 16)]
            for b in range(2 * CB):
                pltpu.async_copy(
                    v_hbm.at[pl.ds(pvidxv[b], 1), :],
                    pv_rows.at[pl.ds(b, 1), :], semp)

        def drain(d):
            pltpu.make_async_copy(
                u_hbm.at[pl.ds(0, UR), :], u_rows.at[d], sems[d]).wait()
            pltpu.make_async_copy(
                v_hbm.at[pl.ds(0, NR), :], n_rows.at[d], sems[d]).wait()

        def drain_pv():
            pltpu.make_async_copy(
                v_hbm.at[pl.ds(0, 2 * CB), :], pv_rows, semp).wait()

        def compute(c, d):
            for b in range(CB):
                acc = [u_rows[d, b * CTX, j * 16:(j + 1) * 16]
                       for j in range(4)]
                for r in range(1, CTX):
                    for j in range(4):
                        acc[j] = acc[j] + u_rows[d, b * CTX + r,
                                                 j * 16:(j + 1) * 16]
                t = acc[0] * pv_rows[d * CB + b, 0:16]
                for j in range(1, 4):
                    t = t + acc[j] * pv_rows[d * CB + b, j * 16:(j + 1) * 16]
                pos_part[pl.ds((c * CB + b) * 16, 16)] = t
                for n in range(NEG):
                    row = b * NEG + n
                    t2 = acc[0] * n_rows[d, row, 0:16]
                    for j in range(1, 4):
                        t2 = t2 + acc[j] * n_rows[d, row, j * 16:(j + 1) * 16]
                    neg_part[row * 16:(row + 1) * 16] = t2
            pltpu.sync_copy(
                neg_part, neg_out.at[pl.ds((base * NEG + c * NR) * 16, NR * 16)])

        fire(0, 0)
        fire_pv(0)

        def body(i2, carry):
            c0 = i2 * 2
            drain(0)
            drain_pv()
            fire(c0 + 1, 1)
            compute(c0, 0)
            drain(1)
            fire(c0 + 2, 0)
            compute(c0 + 1, 1)
            return carry

        def outer(i2, carry):
            carry = body(i2, carry)
            fire_pv(i2 + 1)
            return carry

        lax.fori_loop(0, NCH // 2 - 1, outer, 0)
        body(NCH // 2 - 1, 0)
        pltpu.make_async_copy(
            u_hbm.at[pl.ds(0, UR), :], u_rows.at[0], sems[0]).wait()
        pltpu.make_async_copy(
            v_hbm.at[pl.ds(0, NR), :], n_rows.at[0], sems[0]).wait()
        pltpu.sync_copy(pos_part, pos_out.at[pl.ds(base * 16, BW * 16)])

    return sc_partials


def _make_loss_kernel(n_blocks):
    def loss_kernel(scale_ref, pos_ref, neg_ref, out_ref):
        i = pl.program_id(0)
        inv_len = scale_ref[0]
        rows = lax.broadcasted_iota(jnp.int32, (128, 8), 0)
        cols = lax.broadcasted_iota(jnp.int32, (128, 8), 1)
        fold = jnp.where(rows // 16 == cols, 1.0, 0.0).astype(jnp.float32)

        def logsig(x):
            return jnp.minimum(x, 0.0) - jnp.log1p(jnp.exp(-jnp.abs(x)))

        p = jax.lax.dot(pos_ref[...], fold) * inv_len
        n = jax.lax.dot(neg_ref[...], fold) * inv_len
        part = jnp.sum(logsig(p)) + jnp.sum(logsig(-n))

        @pl.when(i == 0)
        def _():
            out_ref[...] = jnp.zeros((1, 1), jnp.float32)
        out_ref[...] += part[None, None]

    return loss_kernel


def kernel(u_table, v_table, pos_u, pos_v, neg_v, length, embedding_dim):
    B, CTX = pos_u.shape
    NEG = neg_v.shape[1]
    D = u_table.shape[1]
    NW = 32

    posu_flat = jnp.maximum(pos_u.astype(jnp.int32), 0).reshape(-1)
    negv_flat = jnp.maximum(neg_v.astype(jnp.int32), 0).reshape(-1)
    posv = jnp.maximum(pos_v.astype(jnp.int32), 0)

    sc_partials = _make_sc_partials(B, CTX, NEG, D, NW)
    pos_part, neg_part = sc_partials(u_table, v_table, posu_flat, posv, negv_flat)

    pos2d = pos_part.reshape(B * 16 // 128, 128)
    neg2d = neg_part.reshape(B * NEG * 16 // 128, 128)
    GRID = 8
    rp = pos2d.shape[0] // GRID
    rn = neg2d.shape[0] // GRID

    inv_len = (1.0 / jnp.asarray(length, jnp.float32)).reshape(1)

    total = pl.pallas_call(
        _make_loss_kernel(GRID),
        grid=(GRID,),
        in_specs=[
            pl.BlockSpec(memory_space=pltpu.SMEM),
            pl.BlockSpec((rp, 128), lambda i: (i, 0)),
            pl.BlockSpec((rn, 128), lambda i: (i, 0)),
        ],
        out_specs=pl.BlockSpec((1, 1), lambda i: (0, 0)),
        out_shape=jax.ShapeDtypeStruct((1, 1), jnp.float32),
    )(inv_len, pos2d, neg2d)

    return (-total[0, 0]) / jnp.asarray(embedding_dim, jnp.float32)
